# baseline (speedup 1.0000x reference)
import functools

import jax
import jax.numpy as jnp
from jax.experimental import pallas as pl
from jax.experimental.pallas import tpu as pltpu

_CAPACITY = 0.55
_TEMPERATURE = 1.0
_LB_W, _Z_W, _ENT_W = 0.01, 0.0001, 0.001


def _fused_kernel(k, n, h_ref, w_ref, gblk_ref, amblk_ref, amf_ref, b_ref,
                  mask_ref, aux_ref, k_scr, acc_ref):
    i = pl.program_id(0)
    tblk = h_ref.shape[0]

    @pl.when(i == 0)
    def _init():
        acc_ref[0] = 0.0
        acc_ref[1] = 0.0
        acc_ref[2] = 0.0
        acc_ref[3] = 0.0

    s_blk = jax.lax.dot_general(
        h_ref[...], w_ref[...], (((1,), (0,)), ((), ())),
        preferred_element_type=jnp.float32)
    amb = amblk_ref[...].reshape(1, tblk)
    s = s_blk.reshape(1, tblk) + b_ref[0, 0]
    s = jnp.where(amb > 0.0, s, jnp.float32(-1e9))
    noisy = s + gblk_ref[...].reshape(1, tblk) * jnp.float32(_TEMPERATURE)

    bits = jax.lax.bitcast_convert_type(noisy, jnp.int32)
    key = bits ^ jnp.where(bits < 0, jnp.int32(0x7FFFFFFF), jnp.int32(0))
    k_scr[pl.ds(i, 1)] = key.reshape(1, 1, tblk)

    probs = jax.nn.sigmoid(s)
    p = jnp.clip(probs, 1e-4, 1.0 - 1e-4)
    ent = p * jnp.log(p) + (1.0 - p) * jnp.log(1.0 - p)
    acc_ref[0] += jnp.sum(probs * amb)
    acc_ref[1] += jnp.sum(s * s * amb)
    acc_ref[2] += jnp.sum(ent * amb)
    acc_ref[3] += jnp.sum(amb)

    @pl.when(i == pl.num_programs(0) - 1)
    def _route():
        nb = k_scr.shape[0]
        rows, cols = nb, tblk
        key = k_scr[...].reshape(rows, cols)
        kf = jnp.float32(k)

        def thr_body(_, lh):
            lo, hi = lh
            mid = (lo >> 1) + (hi >> 1) + (lo & hi & 1)
            cnt = jnp.sum(jnp.where(key >= mid, 1.0, 0.0))
            ok = cnt >= kf
            return (jnp.where(ok, mid, lo), jnp.where(ok, hi, mid))

        lo, _ = jax.lax.fori_loop(
            0, 32, thr_body,
            (jnp.int32(-2147483648), jnp.int32(2147483647)))
        thr = lo

        gt = key > thr
        cnt_gt = jnp.sum(jnp.where(gt, 1.0, 0.0))
        need = kf - cnt_gt
        eq = key == thr

        ridx = jax.lax.broadcasted_iota(jnp.int32, (rows, cols), 0)
        cidx = jax.lax.broadcasted_iota(jnp.int32, (rows, cols), 1)
        idx = ridx * jnp.int32(cols) + cidx

        def tie_body(_, lh):
            lo2, hi2 = lh
            mid = (lo2 + hi2) >> 1
            c = jnp.sum(jnp.where(eq & (idx < mid), 1.0, 0.0))
            ok = c >= need
            return (jnp.where(ok, lo2, mid), jnp.where(ok, mid, hi2))

        _, m = jax.lax.fori_loop(
            0, 16, tie_body, (jnp.int32(0), jnp.int32(n)))

        sel = (gt | (eq & (idx < m))) & (amf_ref[...] > 0.0)
        mask_ref[...] = sel.astype(jnp.uint8)

        denom = jnp.maximum(acc_ref[3], 1.0)
        soft_fraction = acc_ref[0] / denom
        lb = (soft_fraction - jnp.float32(_CAPACITY)) ** 2
        z = acc_ref[1] / denom
        ent_neg = acc_ref[2] / denom

        def n2n(x):
            return jnp.where(jnp.abs(x) < jnp.inf, x, 0.0)

        aux = (jnp.float32(_LB_W) * n2n(lb) + jnp.float32(_Z_W) * n2n(z)
               + jnp.float32(_ENT_W) * n2n(ent_neg))
        aux_ref[...] = aux.reshape(1, 1)


def kernel(hidden_states, active_mask, W, b):
    B, S, H = hidden_states.shape
    n = B * S
    k = max(1, min(int(_CAPACITY * n + 0.5), n))
    tblk = 4096
    cols = tblk
    rows = n // cols

    hs = hidden_states.astype(jnp.float32).reshape(n, H)

    nkey = jax.random.key(42)
    u = jnp.clip(jax.random.uniform(nkey, (B, S), dtype=jnp.float32),
                 1e-6, 1.0 - 1e-6)
    gumbel = jnp.clip(-jnp.log(-jnp.log(u) + 1e-6), -10.0, 10.0)

    am3 = active_mask.reshape(rows, 1, cols).astype(jnp.float32)
    amf = active_mask.reshape(rows, cols).astype(jnp.float32)
    g3 = gumbel.reshape(rows, 1, cols)
    b2 = b.reshape(1, 1).astype(jnp.float32)

    mask_u8, aux = pl.pallas_call(
        functools.partial(_fused_kernel, k, n),
        grid=(n // tblk,),
        in_specs=[
            pl.BlockSpec((tblk, H), lambda i: (i, 0)),
            pl.BlockSpec((H, 1), lambda i: (0, 0)),
            pl.BlockSpec((1, 1, cols), lambda i: (i, 0, 0)),
            pl.BlockSpec((1, 1, cols), lambda i: (i, 0, 0)),
            pl.BlockSpec((rows, cols), lambda i: (0, 0)),
            pl.BlockSpec((1, 1), lambda i: (0, 0)),
        ],
        out_specs=(
            pl.BlockSpec((rows, cols), lambda i: (0, 0)),
            pl.BlockSpec((1, 1), lambda i: (0, 0)),
        ),
        out_shape=(
            jax.ShapeDtypeStruct((rows, cols), jnp.uint8),
            jax.ShapeDtypeStruct((1, 1), jnp.float32),
        ),
        scratch_shapes=[
            pltpu.VMEM((rows, 1, cols), jnp.int32),
            pltpu.SMEM((4,), jnp.float32),
        ],
    )(hs, W.reshape(H, 1), g3, am3, amf, b2)

    ffn_mask = mask_u8.reshape(B, S).astype(bool)
    return (ffn_mask, aux[0, 0])


# speedup vs baseline: 1.2113x; 1.2113x over previous
import functools

import jax
import jax.numpy as jnp
from jax.experimental import pallas as pl
from jax.experimental.pallas import tpu as pltpu

_CAPACITY = 0.55
_TEMPERATURE = 1.0
_LB_W, _Z_W, _ENT_W = 0.01, 0.0001, 0.001


def _fused_kernel(k, n, h_ref, w_ref, g_ref, am_ref, b_ref, mask_ref,
                  aux_ref, s_scr):
    i = pl.program_id(0)
    tblk = h_ref.shape[0]
    s_blk = jax.lax.dot_general(
        h_ref[...], w_ref[...], (((1,), (0,)), ((), ())),
        preferred_element_type=jnp.float32)
    s_scr[pl.ds(i, 1)] = s_blk.reshape(1, 1, tblk)

    @pl.when(i == pl.num_programs(0) - 1)
    def _route():
        _route_body(k, n, s_scr, g_ref, am_ref, b_ref, mask_ref, aux_ref)


def _route_body(k, n, s_ref, g_ref, am_ref, b_ref, mask_ref, aux_ref):
    nb, _, tblk = s_ref.shape
    rows, cols = nb, tblk
    am = am_ref[...]
    s = s_ref[...].reshape(rows, cols) + b_ref[0, 0]
    s = jnp.where(am > 0.0, s, jnp.float32(-1e9))
    noisy = s + g_ref[...] * jnp.float32(_TEMPERATURE)

    bits = jax.lax.bitcast_convert_type(noisy, jnp.int32)
    key = bits ^ jnp.where(bits < 0, jnp.int32(0x7FFFFFFF), jnp.int32(0))

    kf = jnp.float32(k)

    def thr_body(_, lh):
        lo, hi = lh
        mid = (lo >> 1) + (hi >> 1) + (lo & hi & 1)
        cnt = jnp.sum(jnp.where(key >= mid, 1.0, 0.0))
        ok = cnt >= kf
        return (jnp.where(ok, mid, lo), jnp.where(ok, hi, mid))

    lo, hi = jax.lax.fori_loop(
        0, 2, thr_body,
        (jnp.int32(-2147483648), jnp.int32(2147483647)))

    def thr8_body(_, lh):
        lo, hi = lh
        d = (hi - lo) >> 3
        nlo, nhi = lo, hi
        for j in range(7):
            mid = lo + d * (j + 1)
            cnt = jnp.sum(jnp.where(key >= mid, 1.0, 0.0))
            ok = cnt >= kf
            nlo = jnp.where(ok, jnp.maximum(nlo, mid), nlo)
            nhi = jnp.where(ok, nhi, jnp.minimum(nhi, mid))
        return (nlo, nhi)

    lo, hi = jax.lax.fori_loop(0, 10, thr8_body, (lo, hi))
    lo, _ = jax.lax.fori_loop(0, 4, thr_body, (lo, hi))
    thr = lo

    gt = key > thr
    cnt_gt = jnp.sum(jnp.where(gt, 1.0, 0.0))
    need = kf - cnt_gt
    eq = key == thr

    ridx = jax.lax.broadcasted_iota(jnp.int32, (rows, cols), 0)
    cidx = jax.lax.broadcasted_iota(jnp.int32, (rows, cols), 1)
    idx = ridx * jnp.int32(cols) + cidx

    def tie_body(_, lh):
        lo2, hi2 = lh
        mid = (lo2 + hi2) >> 1
        c = jnp.sum(jnp.where(eq & (idx < mid), 1.0, 0.0))
        ok = c >= need
        return (jnp.where(ok, lo2, mid), jnp.where(ok, mid, hi2))

    def tie8_body(_, lh):
        lo2, hi2 = lh
        d = (hi2 - lo2) >> 3
        nlo, nhi = lo2, hi2
        for j in range(7):
            mid = lo2 + d * (j + 1)
            c = jnp.sum(jnp.where(eq & (idx < mid), 1.0, 0.0))
            ok = c >= need
            nhi = jnp.where(ok, jnp.minimum(nhi, mid), nhi)
            nlo = jnp.where(ok, nlo, jnp.maximum(nlo, mid))
        return (nlo, nhi)

    lo2, hi2 = jax.lax.fori_loop(
        0, 5, tie8_body, (jnp.int32(0), jnp.int32(n)))
    _, m = jax.lax.fori_loop(0, 4, tie_body, (lo2, hi2))

    sel = (gt | (eq & (idx < m))) & (am > 0.0)
    mask_ref[...] = sel.astype(jnp.uint8)

    denom = jnp.maximum(jnp.sum(am), 1.0)
    probs = jax.nn.sigmoid(s)
    soft_fraction = jnp.sum(probs * am) / denom
    lb = (soft_fraction - jnp.float32(_CAPACITY)) ** 2
    z = jnp.sum(s * s * am) / denom
    p = jnp.clip(probs, 1e-4, 1.0 - 1e-4)
    ent_neg = jnp.sum((p * jnp.log(p) + (1.0 - p) * jnp.log(1.0 - p)) * am) / denom

    def n2n(x):
        return jnp.where(jnp.abs(x) < jnp.inf, x, 0.0)

    aux = (jnp.float32(_LB_W) * n2n(lb) + jnp.float32(_Z_W) * n2n(z)
           + jnp.float32(_ENT_W) * n2n(ent_neg))
    aux_ref[...] = aux.reshape(1, 1)


def kernel(hidden_states, active_mask, W, b):
    B, S, H = hidden_states.shape
    n = B * S
    k = max(1, min(int(_CAPACITY * n + 0.5), n))
    tblk = 4096
    cols = tblk
    rows = n // cols

    hs = hidden_states.astype(jnp.float32).reshape(n, H)

    nkey = jax.random.key(42)
    u = jnp.clip(jax.random.uniform(nkey, (B, S), dtype=jnp.float32),
                 1e-6, 1.0 - 1e-6)
    gumbel = jnp.clip(-jnp.log(-jnp.log(u) + 1e-6), -10.0, 10.0)

    am = active_mask.reshape(rows, cols).astype(jnp.float32)
    g = gumbel.reshape(rows, cols)
    b2 = b.reshape(1, 1).astype(jnp.float32)

    mask_u8, aux = pl.pallas_call(
        functools.partial(_fused_kernel, k, n),
        grid=(n // tblk,),
        in_specs=[
            pl.BlockSpec((tblk, H), lambda i: (i, 0)),
            pl.BlockSpec((H, 1), lambda i: (0, 0)),
            pl.BlockSpec((rows, cols), lambda i: (0, 0)),
            pl.BlockSpec((rows, cols), lambda i: (0, 0)),
            pl.BlockSpec((1, 1), lambda i: (0, 0)),
        ],
        out_specs=(
            pl.BlockSpec((rows, cols), lambda i: (0, 0)),
            pl.BlockSpec((1, 1), lambda i: (0, 0)),
        ),
        out_shape=(
            jax.ShapeDtypeStruct((rows, cols), jnp.uint8),
            jax.ShapeDtypeStruct((1, 1), jnp.float32),
        ),
        scratch_shapes=[pltpu.VMEM((n // tblk, 1, tblk), jnp.float32)],
    )(hs, W.reshape(H, 1), g, am, b2)

    ffn_mask = mask_u8.reshape(B, S).astype(bool)
    return (ffn_mask, aux[0, 0])
